# Initial kernel scaffold; baseline (speedup 1.0000x reference)
import jax
import jax.numpy as jnp
from jax.experimental import pallas as pl


def kernel(x, edge_index, edge_attr, W1, b1, W2, b2):
    raise NotImplementedError("write your pallas kernel here")


# speedup vs baseline: 9.6328x; 9.6328x over previous
import functools

import jax
import jax.numpy as jnp
from jax import lax
from jax.experimental import pallas as pl
from jax.experimental.pallas import tpu as pltpu
from jax.experimental.pallas import tpu_sc as plsc

N = 10000
E = 320000
D_IN = 128
D_HID = 256
D_OUT = 128

NC = 2
NS = 16
NW = NC * NS
EB = 128
NPAD = 10240
ROWS_PER_TILE = NPAD // NS
EPW = ((E + NW * EB - 1) // (NW * EB)) * EB
EPAD = EPW * NW
NBATCH = EPW // EB

_mesh = plsc.VectorSubcoreMesh(
    core_axis_name="c", subcore_axis_name="s", num_cores=NC, num_subcores=NS
)


@functools.partial(
    pl.kernel,
    out_type=jax.ShapeDtypeStruct((NC, NPAD), jnp.float32),
    mesh=_mesh,
    scratch_types=[
        pltpu.VMEM((EB,), jnp.int32),
        pltpu.VMEM((EB,), jnp.float32),
        pltpu.VMEM((ROWS_PER_TILE,), jnp.float32),
        pltpu.VMEM_SHARED((NPAD,), jnp.float32),
    ],
)
def _deg_kernel(dst_hbm, w_hbm, out_hbm, dst_v, w_v, zeros_v, deg_sh):
    cid = lax.axis_index("c")
    sid = lax.axis_index("s")
    wid = sid * NC + cid

    def zero_body(i, _):
        zeros_v[pl.ds(i * 16, 16)] = jnp.zeros((16,), jnp.float32)
        return _

    lax.fori_loop(0, ROWS_PER_TILE // 16, zero_body, 0)
    pltpu.sync_copy(zeros_v, deg_sh.at[pl.ds(sid * ROWS_PER_TILE, ROWS_PER_TILE)])
    plsc.subcore_barrier()

    def batch_body(b, _):
        base = wid * EPW + b * EB
        pltpu.sync_copy(dst_hbm.at[pl.ds(base, EB)], dst_v)
        pltpu.sync_copy(w_hbm.at[pl.ds(base, EB)], w_v)
        pltpu.sync_copy(w_v, deg_sh.at[dst_v], add=True)
        return _

    lax.fori_loop(0, NBATCH, batch_body, 0)
    plsc.subcore_barrier()
    pltpu.sync_copy(
        deg_sh.at[pl.ds(sid * ROWS_PER_TILE, ROWS_PER_TILE)],
        out_hbm.at[cid, pl.ds(sid * ROWS_PER_TILE, ROWS_PER_TILE)],
    )


@functools.partial(
    pl.kernel,
    out_type=jax.ShapeDtypeStruct((NC, NPAD, D_IN), jnp.float32),
    mesh=_mesh,
    scratch_types=[
        pltpu.VMEM((EB,), jnp.int32),
        pltpu.VMEM((EB,), jnp.int32),
        pltpu.VMEM((EB,), jnp.float32),
        pltpu.VMEM((EB, D_IN), jnp.float32),
        pltpu.VMEM_SHARED((NPAD, D_IN), jnp.float32),
    ],
)
def _agg_kernel(vals_hbm, src_hbm, dst_hbm, w_hbm, out_hbm,
                src_v, dst_v, w_v, rows_v, acc_sh):
    cid = lax.axis_index("c")
    sid = lax.axis_index("s")
    wid = sid * NC + cid

    def zero_body(i, _):
        r = i // (D_IN // 16)
        c = i % (D_IN // 16)
        rows_v[r, pl.ds(c * 16, 16)] = jnp.zeros((16,), jnp.float32)
        return _

    lax.fori_loop(0, EB * (D_IN // 16), zero_body, 0)
    for k in range(ROWS_PER_TILE // EB):
        pltpu.sync_copy(
            rows_v, acc_sh.at[pl.ds(sid * ROWS_PER_TILE + k * EB, EB)]
        )
    plsc.subcore_barrier()

    def batch_body(b, _):
        base = wid * EPW + b * EB
        pltpu.sync_copy(src_hbm.at[pl.ds(base, EB)], src_v)
        pltpu.sync_copy(dst_hbm.at[pl.ds(base, EB)], dst_v)
        pltpu.sync_copy(w_hbm.at[pl.ds(base, EB)], w_v)
        pltpu.sync_copy(vals_hbm.at[src_v], rows_v)

        def scale_body(j, _):
            w16 = w_v[pl.ds(j * 16, 16)]
            for k in range(16):
                e = j * 16 + k
                wv = w16[k]
                for c in range(D_IN // 16):
                    rows_v[e, pl.ds(c * 16, 16)] = rows_v[e, pl.ds(c * 16, 16)] * wv
            return _

        lax.fori_loop(0, EB // 16, scale_body, 0)
        pltpu.sync_copy(rows_v, acc_sh.at[dst_v], add=True)
        return _

    lax.fori_loop(0, NBATCH, batch_body, 0)
    plsc.subcore_barrier()
    pltpu.sync_copy(
        acc_sh.at[pl.ds(sid * ROWS_PER_TILE, ROWS_PER_TILE)],
        out_hbm.at[cid, pl.ds(sid * ROWS_PER_TILE, ROWS_PER_TILE)],
    )


BLK = 1000


def _scale_body(dpt_ref, x_ref, dis_ref, xs_ref):
    deg = dpt_ref[:, 0:1] + dpt_ref[:, 1:2] + 1.0
    dis = lax.rsqrt(deg)
    dis_ref[...] = dis
    xs_ref[...] = x_ref[...] * dis


def _mlp_body(p_ref, dis_ref, xs_ref, w1_ref, b1_ref, w2_ref, xs2_ref):
    dis = dis_ref[...]
    agg = p_ref[0] + p_ref[1]
    r1 = dis * (agg + xs_ref[...])
    h = jnp.dot(r1, w1_ref[...], preferred_element_type=jnp.float32) + b1_ref[...]
    h = jnp.maximum(h, 0.0)
    g = jnp.dot(h, w2_ref[...], preferred_element_type=jnp.float32)
    xs2_ref[...] = dis * g


def _final_body(p_ref, dis_ref, xs2_ref, b2_ref, out_ref):
    dis = dis_ref[...]
    agg = p_ref[0] + p_ref[1]
    out_ref[...] = dis * (agg + xs2_ref[...]) + b2_ref[...]


def kernel(x, edge_index, edge_attr, W1, b1, W2, b2):
    f32 = jnp.float32
    src = edge_index[0]
    dst = edge_index[1]
    pad = EPAD - E
    src_p = jnp.concatenate([src, jnp.zeros((pad,), jnp.int32)])
    dst_p = jnp.concatenate([dst, jnp.zeros((pad,), jnp.int32)])
    w_p = jnp.concatenate([edge_attr, jnp.zeros((pad,), f32)])

    deg_parts = _deg_kernel(dst_p, w_p)
    dpt = jnp.transpose(deg_parts)[:N]

    grid = N // BLK
    dis, xs1 = pl.pallas_call(
        _scale_body,
        grid=(grid,),
        in_specs=[
            pl.BlockSpec((BLK, NC), lambda i: (i, 0)),
            pl.BlockSpec((BLK, D_IN), lambda i: (i, 0)),
        ],
        out_specs=[
            pl.BlockSpec((BLK, 1), lambda i: (i, 0)),
            pl.BlockSpec((BLK, D_IN), lambda i: (i, 0)),
        ],
        out_shape=[
            jax.ShapeDtypeStruct((N, 1), f32),
            jax.ShapeDtypeStruct((N, D_IN), f32),
        ],
    )(dpt, x)

    parts1 = _agg_kernel(xs1, src_p, dst_p, w_p)

    xs2 = pl.pallas_call(
        _mlp_body,
        grid=(grid,),
        in_specs=[
            pl.BlockSpec((NC, BLK, D_IN), lambda i: (0, i, 0)),
            pl.BlockSpec((BLK, 1), lambda i: (i, 0)),
            pl.BlockSpec((BLK, D_IN), lambda i: (i, 0)),
            pl.BlockSpec((D_IN, D_HID), lambda i: (0, 0)),
            pl.BlockSpec((1, D_HID), lambda i: (0, 0)),
            pl.BlockSpec((D_HID, D_OUT), lambda i: (0, 0)),
        ],
        out_specs=pl.BlockSpec((BLK, D_OUT), lambda i: (i, 0)),
        out_shape=jax.ShapeDtypeStruct((N, D_OUT), f32),
    )(parts1, dis, xs1, W1, b1.reshape(1, D_HID), W2)

    parts2 = _agg_kernel(xs2, src_p, dst_p, w_p)

    out = pl.pallas_call(
        _final_body,
        grid=(grid,),
        in_specs=[
            pl.BlockSpec((NC, BLK, D_OUT), lambda i: (0, i, 0)),
            pl.BlockSpec((BLK, 1), lambda i: (i, 0)),
            pl.BlockSpec((BLK, D_OUT), lambda i: (i, 0)),
            pl.BlockSpec((1, D_OUT), lambda i: (0, 0)),
        ],
        out_specs=pl.BlockSpec((BLK, D_OUT), lambda i: (i, 0)),
        out_shape=jax.ShapeDtypeStruct((N, D_OUT), f32),
    )(parts2, dis, xs2, b2.reshape(1, D_OUT))

    return out


# speedup vs baseline: 10.7553x; 1.1165x over previous
import functools

import jax
import jax.numpy as jnp
from jax import lax
from jax.experimental import pallas as pl
from jax.experimental.pallas import tpu as pltpu
from jax.experimental.pallas import tpu_sc as plsc

N = 10000
E = 320000
D_IN = 128
D_HID = 256
D_OUT = 128

NC = 2
NS = 16
NW = NC * NS
EB = 64
NBUF = 4
NRB = 3
CH = 40
NCHUNK = 4
NPAD = 10240
ROWS_PER_TILE = NPAD // NS
NBATCH = CH * NCHUNK
EPW = NBATCH * EB
EPAD = EPW * NW

_mesh = plsc.VectorSubcoreMesh(
    core_axis_name="c", subcore_axis_name="s", num_cores=NC, num_subcores=NS
)


@functools.partial(
    pl.kernel,
    out_type=jax.ShapeDtypeStruct((NC, NPAD), jnp.float32),
    mesh=_mesh,
    scratch_types=[
        pltpu.VMEM((NBATCH, EB), jnp.int32),
        pltpu.VMEM((NBATCH, EB), jnp.float32),
        pltpu.VMEM((ROWS_PER_TILE,), jnp.float32),
        [pltpu.SemaphoreType.DMA for _ in range(NBUF)],
        pltpu.VMEM_SHARED((NPAD,), jnp.float32),
    ],
)
def _deg_kernel(dst_hbm, w_hbm, out_hbm, dst_v, w_v, zeros_v, sems, deg_sh):
    cid = lax.axis_index("c")
    sid = lax.axis_index("s")
    wid = sid * NC + cid

    def zero_body(i, _):
        zeros_v[pl.ds(i * 16, 16)] = jnp.zeros((16,), jnp.float32)
        return _

    lax.fori_loop(0, ROWS_PER_TILE // 16, zero_body, 0)
    pltpu.sync_copy(zeros_v, deg_sh.at[pl.ds(sid * ROWS_PER_TILE, ROWS_PER_TILE)])
    pltpu.sync_copy(dst_hbm.at[wid], dst_v)
    pltpu.sync_copy(w_hbm.at[wid], w_v)
    plsc.subcore_barrier()

    def issue(b, f):
        pltpu.async_copy(w_v.at[b], deg_sh.at[dst_v.at[b]], sems[f], add=True)

    def wait(b, f):
        pltpu.make_async_copy(w_v.at[b], deg_sh.at[dst_v.at[b]], sems[f]).wait()

    for k in range(NBUF):
        issue(k, k)

    def ring_body(i, _):
        for k in range(NBUF):
            b = (i + 1) * NBUF + k
            wait(b - NBUF, k)
            issue(b, k)
        return _

    lax.fori_loop(0, NBATCH // NBUF - 1, ring_body, 0)
    for k in range(NBUF):
        wait(NBATCH - NBUF + k, k)
    plsc.subcore_barrier()
    pltpu.sync_copy(
        deg_sh.at[pl.ds(sid * ROWS_PER_TILE, ROWS_PER_TILE)],
        out_hbm.at[cid, pl.ds(sid * ROWS_PER_TILE, ROWS_PER_TILE)],
    )


@functools.partial(
    pl.kernel,
    out_type=jax.ShapeDtypeStruct((NC, NPAD, D_IN), jnp.float32),
    mesh=_mesh,
    scratch_types=[
        pltpu.VMEM((CH, EB), jnp.int32),
        pltpu.VMEM((CH, EB), jnp.int32),
        pltpu.VMEM((CH, EB), jnp.float32),
        [pltpu.VMEM((EB, D_IN), jnp.float32) for _ in range(NRB)],
        [pltpu.SemaphoreType.DMA for _ in range(NRB)],
        [pltpu.SemaphoreType.DMA for _ in range(NRB)],
        pltpu.VMEM_SHARED((NPAD, D_IN), jnp.float32),
    ],
)
def _agg_kernel(vals_hbm, src_hbm, dst_hbm, w_hbm, out_hbm,
                src_v, dst_v, w_v, rows, gsem, ssem, acc_sh):
    cid = lax.axis_index("c")
    sid = lax.axis_index("s")
    wid = sid * NC + cid

    def zero_body(i, _):
        r = i // (D_IN // 16)
        c = i % (D_IN // 16)
        rows[0][r, pl.ds(c * 16, 16)] = jnp.zeros((16,), jnp.float32)
        return _

    lax.fori_loop(0, EB * (D_IN // 16), zero_body, 0)
    for k in range(ROWS_PER_TILE // EB):
        pltpu.sync_copy(
            rows[0], acc_sh.at[pl.ds(sid * ROWS_PER_TILE + k * EB, EB)]
        )
    plsc.subcore_barrier()

    def gissue(b, f):
        pltpu.async_copy(vals_hbm.at[src_v.at[b]], rows[f], gsem[f])

    def gwait(b, f):
        pltpu.make_async_copy(vals_hbm.at[src_v.at[b]], rows[f], gsem[f]).wait()

    def sissue(b, f):
        pltpu.async_copy(rows[f], acc_sh.at[dst_v.at[b]], ssem[f], add=True)

    def swait(b, f):
        pltpu.make_async_copy(rows[f], acc_sh.at[dst_v.at[b]], ssem[f]).wait()

    def scale(b, f):
        def scale_body(j, _):
            w16 = w_v[b, pl.ds(j * 16, 16)]
            for k in range(16):
                e = j * 16 + k
                wv = w16[k]
                for c in range(D_IN // 16):
                    rows[f][e, pl.ds(c * 16, 16)] = (
                        rows[f][e, pl.ds(c * 16, 16)] * wv
                    )
            return _

        lax.fori_loop(0, EB // 16, scale_body, 0)

    def chunk_body(c, carry):
        pltpu.sync_copy(src_hbm.at[wid, pl.ds(c * CH, CH)], src_v)
        pltpu.sync_copy(dst_hbm.at[wid, pl.ds(c * CH, CH)], dst_v)
        pltpu.sync_copy(w_hbm.at[wid, pl.ds(c * CH, CH)], w_v)
        gissue(0, 0)
        gissue(1, 1)
        gwait(0, 0)
        scale(0, 0)
        sissue(0, 0)
        gissue(2, 2)
        gwait(1, 1)
        scale(1, 1)
        sissue(1, 1)
        swait(0, 0)
        gissue(3, 0)

        def main_body(i, _):
            b0 = 2 + i * NRB
            for k in range(NRB):
                b = b0 + k
                f_cur = (2 + k) % NRB
                f_pre = (1 + k) % NRB
                gwait(b, f_cur)
                scale(b, f_cur)
                sissue(b, f_cur)
                swait(b - 1, f_pre)
                gissue(b + 2, f_pre)
            return _

        lax.fori_loop(0, (CH - 4) // NRB, main_body, 0)
        for k in (2, 1):
            b = CH - k
            f = b % NRB
            gwait(b, f)
            scale(b, f)
            sissue(b, f)
            swait(b - 1, (b - 1) % NRB)
        swait(CH - 1, (CH - 1) % NRB)
        return carry

    lax.fori_loop(0, NCHUNK, chunk_body, 0)

    plsc.subcore_barrier()
    pltpu.sync_copy(
        acc_sh.at[pl.ds(sid * ROWS_PER_TILE, ROWS_PER_TILE)],
        out_hbm.at[cid, pl.ds(sid * ROWS_PER_TILE, ROWS_PER_TILE)],
    )


BLK = 1000


def _scale_body(dpt_ref, x_ref, dis_ref, xs_ref):
    deg = dpt_ref[:, 0:1] + dpt_ref[:, 1:2] + 1.0
    dis = lax.rsqrt(deg)
    dis_ref[...] = dis
    xs_ref[...] = x_ref[...] * dis


def _mlp_body(p_ref, dis_ref, xs_ref, w1_ref, b1_ref, w2_ref, xs2_ref):
    dis = dis_ref[...]
    agg = p_ref[0] + p_ref[1]
    r1 = dis * (agg + xs_ref[...])
    h = jnp.dot(r1, w1_ref[...], preferred_element_type=jnp.float32) + b1_ref[...]
    h = jnp.maximum(h, 0.0)
    g = jnp.dot(h, w2_ref[...], preferred_element_type=jnp.float32)
    xs2_ref[...] = dis * g


def _final_body(p_ref, dis_ref, xs2_ref, b2_ref, out_ref):
    dis = dis_ref[...]
    agg = p_ref[0] + p_ref[1]
    out_ref[...] = dis * (agg + xs2_ref[...]) + b2_ref[...]


def kernel(x, edge_index, edge_attr, W1, b1, W2, b2):
    f32 = jnp.float32
    src = edge_index[0]
    dst = edge_index[1]
    pad = EPAD - E
    src_p = jnp.concatenate([src, jnp.zeros((pad,), jnp.int32)]).reshape(
        NW, NBATCH, EB
    )
    dst_p = jnp.concatenate([dst, jnp.zeros((pad,), jnp.int32)]).reshape(
        NW, NBATCH, EB
    )
    w_p = jnp.concatenate([edge_attr, jnp.zeros((pad,), f32)]).reshape(
        NW, NBATCH, EB
    )

    deg_parts = _deg_kernel(dst_p, w_p)
    dpt = jnp.transpose(deg_parts)[:N]

    grid = N // BLK
    dis, xs1 = pl.pallas_call(
        _scale_body,
        grid=(grid,),
        in_specs=[
            pl.BlockSpec((BLK, NC), lambda i: (i, 0)),
            pl.BlockSpec((BLK, D_IN), lambda i: (i, 0)),
        ],
        out_specs=[
            pl.BlockSpec((BLK, 1), lambda i: (i, 0)),
            pl.BlockSpec((BLK, D_IN), lambda i: (i, 0)),
        ],
        out_shape=[
            jax.ShapeDtypeStruct((N, 1), f32),
            jax.ShapeDtypeStruct((N, D_IN), f32),
        ],
    )(dpt, x)

    parts1 = _agg_kernel(xs1, src_p, dst_p, w_p)

    xs2 = pl.pallas_call(
        _mlp_body,
        grid=(grid,),
        in_specs=[
            pl.BlockSpec((NC, BLK, D_IN), lambda i: (0, i, 0)),
            pl.BlockSpec((BLK, 1), lambda i: (i, 0)),
            pl.BlockSpec((BLK, D_IN), lambda i: (i, 0)),
            pl.BlockSpec((D_IN, D_HID), lambda i: (0, 0)),
            pl.BlockSpec((1, D_HID), lambda i: (0, 0)),
            pl.BlockSpec((D_HID, D_OUT), lambda i: (0, 0)),
        ],
        out_specs=pl.BlockSpec((BLK, D_OUT), lambda i: (i, 0)),
        out_shape=jax.ShapeDtypeStruct((N, D_OUT), f32),
    )(parts1, dis, xs1, W1, b1.reshape(1, D_HID), W2)

    parts2 = _agg_kernel(xs2, src_p, dst_p, w_p)

    out = pl.pallas_call(
        _final_body,
        grid=(grid,),
        in_specs=[
            pl.BlockSpec((NC, BLK, D_IN), lambda i: (0, i, 0)),
            pl.BlockSpec((BLK, 1), lambda i: (i, 0)),
            pl.BlockSpec((BLK, D_OUT), lambda i: (i, 0)),
            pl.BlockSpec((1, D_OUT), lambda i: (0, 0)),
        ],
        out_specs=pl.BlockSpec((BLK, D_OUT), lambda i: (i, 0)),
        out_shape=jax.ShapeDtypeStruct((N, D_OUT), f32),
    )(parts2, dis, xs2, b2.reshape(1, D_OUT))

    return out


# speedup vs baseline: 29.8400x; 2.7744x over previous
import functools

import jax
import jax.numpy as jnp
from jax import lax
from jax.experimental import pallas as pl
from jax.experimental.pallas import tpu as pltpu
from jax.experimental.pallas import tpu_sc as plsc

N = 10000
E = 320000
D_IN = 128
D_HID = 256
D_OUT = 128

NC = 2
NS = 16
NW = NC * NS
EB = 64
NBUF = 4
NRB = 3
CH = 40
NCHUNK = 4
NPAD = 10240
ROWS_PER_TILE = NPAD // NS
NBATCH = CH * NCHUNK
EPW = NBATCH * EB
EPAD = EPW * NW

_mesh = plsc.VectorSubcoreMesh(
    core_axis_name="c", subcore_axis_name="s", num_cores=NC, num_subcores=NS
)


@functools.partial(
    pl.kernel,
    out_type=jax.ShapeDtypeStruct((NC, NPAD), jnp.float32),
    mesh=_mesh,
    scratch_types=[
        pltpu.VMEM((NBATCH, EB), jnp.int32),
        pltpu.VMEM((NBATCH, EB), jnp.float32),
        pltpu.VMEM((ROWS_PER_TILE,), jnp.float32),
        [pltpu.SemaphoreType.DMA for _ in range(NBUF)],
        pltpu.VMEM_SHARED((NPAD,), jnp.float32),
    ],
)
def _deg_kernel(dst_hbm, w_hbm, out_hbm, dst_v, w_v, zeros_v, sems, deg_sh):
    cid = lax.axis_index("c")
    sid = lax.axis_index("s")
    wid = sid * NC + cid

    def zero_body(i, _):
        zeros_v[pl.ds(i * 16, 16)] = jnp.zeros((16,), jnp.float32)
        return _

    lax.fori_loop(0, ROWS_PER_TILE // 16, zero_body, 0)
    pltpu.sync_copy(zeros_v, deg_sh.at[pl.ds(sid * ROWS_PER_TILE, ROWS_PER_TILE)])
    pltpu.sync_copy(dst_hbm.at[wid], dst_v)
    pltpu.sync_copy(w_hbm.at[wid], w_v)
    plsc.subcore_barrier()

    def issue(b, f):
        pltpu.async_copy(w_v.at[b], deg_sh.at[dst_v.at[b]], sems[f], add=True)

    def wait(b, f):
        pltpu.make_async_copy(w_v.at[b], deg_sh.at[dst_v.at[b]], sems[f]).wait()

    for k in range(NBUF):
        issue(k, k)

    def ring_body(i, _):
        for k in range(NBUF):
            b = (i + 1) * NBUF + k
            wait(b - NBUF, k)
            issue(b, k)
        return _

    lax.fori_loop(0, NBATCH // NBUF - 1, ring_body, 0)
    for k in range(NBUF):
        wait(NBATCH - NBUF + k, k)
    plsc.subcore_barrier()
    pltpu.sync_copy(
        deg_sh.at[pl.ds(sid * ROWS_PER_TILE, ROWS_PER_TILE)],
        out_hbm.at[cid, pl.ds(sid * ROWS_PER_TILE, ROWS_PER_TILE)],
    )


@functools.partial(
    pl.kernel,
    out_type=jax.ShapeDtypeStruct((NC, NPAD, D_IN), jnp.float32),
    mesh=_mesh,
    scratch_types=[
        pltpu.VMEM((CH, EB), jnp.int32),
        pltpu.VMEM((CH, EB), jnp.int32),
        pltpu.VMEM((CH, EB), jnp.float32),
        [pltpu.VMEM((EB, D_IN), jnp.float32) for _ in range(NRB)],
        [pltpu.SemaphoreType.DMA for _ in range(NRB)],
        [pltpu.SemaphoreType.DMA for _ in range(NRB)],
        pltpu.VMEM_SHARED((NPAD, D_IN), jnp.float32),
    ],
)
def _agg_kernel(vals_hbm, src_hbm, dst_hbm, w_hbm, out_hbm,
                src_v, dst_v, w_v, rows, gsem, ssem, acc_sh):
    cid = lax.axis_index("c")
    sid = lax.axis_index("s")
    wid = sid * NC + cid

    def zero_body(i, _):
        r = i // (D_IN // 16)
        c = i % (D_IN // 16)
        rows[0][r, pl.ds(c * 16, 16)] = jnp.zeros((16,), jnp.float32)
        return _

    lax.fori_loop(0, EB * (D_IN // 16), zero_body, 0)
    for k in range(ROWS_PER_TILE // EB):
        pltpu.sync_copy(
            rows[0], acc_sh.at[pl.ds(sid * ROWS_PER_TILE + k * EB, EB)]
        )
    plsc.subcore_barrier()

    def gissue(b, f):
        pltpu.async_copy(vals_hbm.at[src_v.at[b]], rows[f], gsem[f])

    def gwait(b, f):
        pltpu.make_async_copy(vals_hbm.at[src_v.at[b]], rows[f], gsem[f]).wait()

    def sissue(b, f):
        pltpu.async_copy(rows[f], acc_sh.at[dst_v.at[b]], ssem[f], add=True)

    def swait(b, f):
        pltpu.make_async_copy(rows[f], acc_sh.at[dst_v.at[b]], ssem[f]).wait()

    def scale(b, f):
        def scale_body(j, _):
            w16 = w_v[b, pl.ds(j * 16, 16)]
            for k in range(16):
                e = j * 16 + k
                wv = w16[k]
                for c in range(D_IN // 16):
                    rows[f][e, pl.ds(c * 16, 16)] = (
                        rows[f][e, pl.ds(c * 16, 16)] * wv
                    )
            return _

        lax.fori_loop(0, EB // 16, scale_body, 0)

    def chunk_body(c, carry):
        pltpu.sync_copy(src_hbm.at[wid, pl.ds(c * CH, CH)], src_v)
        pltpu.sync_copy(dst_hbm.at[wid, pl.ds(c * CH, CH)], dst_v)
        pltpu.sync_copy(w_hbm.at[wid, pl.ds(c * CH, CH)], w_v)
        gissue(0, 0)
        gissue(1, 1)
        gwait(0, 0)
        scale(0, 0)
        sissue(0, 0)
        gissue(2, 2)
        gwait(1, 1)
        scale(1, 1)
        sissue(1, 1)
        swait(0, 0)
        gissue(3, 0)

        def main_body(i, _):
            b0 = 2 + i * NRB
            for k in range(NRB):
                b = b0 + k
                f_cur = (2 + k) % NRB
                f_pre = (1 + k) % NRB
                gwait(b, f_cur)
                scale(b, f_cur)
                sissue(b, f_cur)
                swait(b - 1, f_pre)
                gissue(b + 2, f_pre)
            return _

        lax.fori_loop(0, (CH - 4) // NRB, main_body, 0)
        for k in (2, 1):
            b = CH - k
            f = b % NRB
            gwait(b, f)
            scale(b, f)
            sissue(b, f)
            swait(b - 1, (b - 1) % NRB)
        swait(CH - 1, (CH - 1) % NRB)
        return carry

    lax.fori_loop(0, NCHUNK, chunk_body, 0)

    plsc.subcore_barrier()
    pltpu.sync_copy(
        acc_sh.at[pl.ds(sid * ROWS_PER_TILE, ROWS_PER_TILE)],
        out_hbm.at[cid, pl.ds(sid * ROWS_PER_TILE, ROWS_PER_TILE)],
    )


BLK = 1000


def _scale_body(dpt_ref, x_ref, dis_ref, xs_ref):
    deg = dpt_ref[:, 0:1] + dpt_ref[:, 1:2] + 1.0
    dis = lax.rsqrt(deg)
    dis_ref[...] = dis
    xs_ref[...] = x_ref[...] * dis


def _mlp_body(p_ref, dis_ref, xs_ref, w1_ref, b1_ref, w2_ref, xs2_ref):
    dis = dis_ref[...]
    agg = p_ref[0] + p_ref[1]
    r1 = dis * (agg + xs_ref[...])
    h = jnp.dot(r1, w1_ref[...], preferred_element_type=jnp.float32) + b1_ref[...]
    h = jnp.maximum(h, 0.0)
    g = jnp.dot(h, w2_ref[...], preferred_element_type=jnp.float32)
    xs2_ref[...] = dis * g


def _final_body(p_ref, dis_ref, xs2_ref, b2_ref, out_ref):
    dis = dis_ref[...]
    agg = p_ref[0] + p_ref[1]
    out_ref[...] = dis * (agg + xs2_ref[...]) + b2_ref[...]


def kernel(x, edge_index, edge_attr, W1, b1, W2, b2):
    f32 = jnp.float32
    src = edge_index[0]
    dst = edge_index[1]
    pad = EPAD - E
    spread = jnp.arange(pad, dtype=jnp.int32) % N
    src_p = jnp.concatenate([src, spread]).reshape(NW, NBATCH, EB)
    dst_p = jnp.concatenate([dst, spread]).reshape(NW, NBATCH, EB)
    w_p = jnp.concatenate([edge_attr, jnp.zeros((pad,), f32)]).reshape(
        NW, NBATCH, EB
    )

    deg_parts = _deg_kernel(dst_p, w_p)
    dpt = jnp.transpose(deg_parts)[:N]

    grid = N // BLK
    dis, xs1 = pl.pallas_call(
        _scale_body,
        grid=(grid,),
        in_specs=[
            pl.BlockSpec((BLK, NC), lambda i: (i, 0)),
            pl.BlockSpec((BLK, D_IN), lambda i: (i, 0)),
        ],
        out_specs=[
            pl.BlockSpec((BLK, 1), lambda i: (i, 0)),
            pl.BlockSpec((BLK, D_IN), lambda i: (i, 0)),
        ],
        out_shape=[
            jax.ShapeDtypeStruct((N, 1), f32),
            jax.ShapeDtypeStruct((N, D_IN), f32),
        ],
    )(dpt, x)

    parts1 = _agg_kernel(xs1, src_p, dst_p, w_p)

    xs2 = pl.pallas_call(
        _mlp_body,
        grid=(grid,),
        in_specs=[
            pl.BlockSpec((NC, BLK, D_IN), lambda i: (0, i, 0)),
            pl.BlockSpec((BLK, 1), lambda i: (i, 0)),
            pl.BlockSpec((BLK, D_IN), lambda i: (i, 0)),
            pl.BlockSpec((D_IN, D_HID), lambda i: (0, 0)),
            pl.BlockSpec((1, D_HID), lambda i: (0, 0)),
            pl.BlockSpec((D_HID, D_OUT), lambda i: (0, 0)),
        ],
        out_specs=pl.BlockSpec((BLK, D_OUT), lambda i: (i, 0)),
        out_shape=jax.ShapeDtypeStruct((N, D_OUT), f32),
    )(parts1, dis, xs1, W1, b1.reshape(1, D_HID), W2)

    parts2 = _agg_kernel(xs2, src_p, dst_p, w_p)

    out = pl.pallas_call(
        _final_body,
        grid=(grid,),
        in_specs=[
            pl.BlockSpec((NC, BLK, D_IN), lambda i: (0, i, 0)),
            pl.BlockSpec((BLK, 1), lambda i: (i, 0)),
            pl.BlockSpec((BLK, D_OUT), lambda i: (i, 0)),
            pl.BlockSpec((1, D_OUT), lambda i: (0, 0)),
        ],
        out_specs=pl.BlockSpec((BLK, D_OUT), lambda i: (i, 0)),
        out_shape=jax.ShapeDtypeStruct((N, D_OUT), f32),
    )(parts2, dis, xs2, b2.reshape(1, D_OUT))

    return out
